# baseline (speedup 1.0000x reference)
import jax
import jax.numpy as jnp
from jax import lax
from jax.experimental import pallas as pl
from jax.experimental.pallas import tpu as pltpu
from jax.experimental.pallas import tpu_sc as plsc

BATCH = 16384
N_FIELDS = 26
NUM_FACTORS = 16

NC = 2
NS = 16
NW = NC * NS
ROWS_PER_W = BATCH // NW
IDX_PER_W = ROWS_PER_W * N_FIELDS
GATHER_W = 128
IDX_ROWS_PER_W = IDX_PER_W // GATHER_W
CHUNK_ROWS = 64
CHUNK_IDX = CHUNK_ROWS * N_FIELDS
DMAS_PER_CHUNK = CHUNK_IDX // GATHER_W
N_CHUNKS = ROWS_PER_W // CHUNK_ROWS


def _fm_body(x_hbm, emb_hbm, fc_hbm, out_hbm, fcout_hbm,
             idx_v, emb_buf, fc_buf, out_v, fcvec_v, sem):
    wid = lax.axis_index("s") * NC + lax.axis_index("c")

    pltpu.sync_copy(x_hbm.at[pl.ds(wid * IDX_ROWS_PER_W, IDX_ROWS_PER_W), :],
                    idx_v)

    def start_chunk(g, ebuf, fbuf):
        handles = []
        for j in range(DMAS_PER_CHUNK):
            isl = idx_v.at[g * DMAS_PER_CHUNK + j]
            handles.append(pltpu.async_copy(
                emb_hbm.at[isl], ebuf.at[pl.ds(j * GATHER_W, GATHER_W), :], sem))
            handles.append(pltpu.async_copy(
                fc_hbm.at[isl], fbuf.at[pl.ds(j * GATHER_W, GATHER_W)], sem))
        return handles

    lane_ids = lax.iota(jnp.int32, NUM_FACTORS)

    def hsum(v):
        for k in (8, 4, 2, 1):
            v = v + v.at[lane_ids ^ k].get(mode="promise_in_bounds")
        return v

    def compute_chunk(g, ebuf, fbuf, fc_acc):
        out_base = g * CHUNK_ROWS

        def pair_body(i, acc):
            r = 2 * i
            inters = []
            for dr in (0, 1):
                rbase = (r + dr) * N_FIELDS
                s = ebuf[rbase, :]
                sq = s * s
                for j in range(1, N_FIELDS):
                    v = ebuf[rbase + j, :]
                    s = s + v
                    sq = sq + v * v
                inters.append(s * s - sq)
            h0, h1 = hsum(inters[0]), hsum(inters[1])
            lane = r % NUM_FACTORS
            acc = jnp.where(lane_ids == lane, 0.5 * h0, acc)
            acc = jnp.where(lane_ids == lane + 1, 0.5 * h1, acc)

            @pl.when(lane == NUM_FACTORS - 2)
            def _flush():
                out_v[pl.ds(out_base + r - (NUM_FACTORS - 2), NUM_FACTORS)] = acc

            return acc

        lax.fori_loop(0, CHUNK_ROWS // 2, pair_body,
                      jnp.zeros((NUM_FACTORS,), jnp.float32))

        return lax.fori_loop(
            0, CHUNK_IDX // NUM_FACTORS,
            lambda k, a: a + fbuf[pl.ds(k * NUM_FACTORS, NUM_FACTORS)],
            fc_acc)

    fc_acc = jnp.zeros((NUM_FACTORS,), jnp.float32)
    handles = start_chunk(0, emb_buf.at[0], fc_buf.at[0])
    for g in range(N_CHUNKS):
        p = g % 2
        for h in handles:
            h.wait()
        if g + 1 < N_CHUNKS:
            handles = start_chunk(g + 1, emb_buf.at[1 - p], fc_buf.at[1 - p])
        else:
            handles = []
        fc_acc = compute_chunk(g, emb_buf.at[p], fc_buf.at[p], fc_acc)

    fcvec_v[:] = fc_acc
    pltpu.sync_copy(out_v, out_hbm.at[pl.ds(wid * ROWS_PER_W, ROWS_PER_W)])
    pltpu.sync_copy(fcvec_v, fcout_hbm.at[wid])


def kernel(X, emb_table, fc_table, dense_W, dense_b):
    x_flat = X.astype(jnp.int32).reshape(BATCH * N_FIELDS // GATHER_W, GATHER_W)
    fc_flat = fc_table.reshape(-1)

    mesh = plsc.VectorSubcoreMesh(core_axis_name="c", subcore_axis_name="s")
    fm = pl.kernel(
        _fm_body,
        mesh=mesh,
        compiler_params=pltpu.CompilerParams(
            use_tc_tiling_on_sc=False,
            disable_bounds_checks=True,
            disable_semaphore_checks=True,
        ),
        out_type=[
            jax.ShapeDtypeStruct((BATCH,), jnp.float32),
            jax.ShapeDtypeStruct((NW, NUM_FACTORS), jnp.float32),
        ],
        scratch_types=[
            pltpu.VMEM((IDX_ROWS_PER_W, GATHER_W), jnp.int32),
            pltpu.VMEM((2, CHUNK_IDX, NUM_FACTORS), jnp.float32),
            pltpu.VMEM((2, CHUNK_IDX), jnp.float32),
            pltpu.VMEM((ROWS_PER_W,), jnp.float32),
            pltpu.VMEM((NUM_FACTORS,), jnp.float32),
            pltpu.SemaphoreType.DMA,
        ],
    )
    inter_half, fc_parts = fm(x_flat, emb_table, fc_flat)

    linear_term = (dense_W[0, 0] * jnp.sum(fc_parts)
                   + dense_b[0] * (BATCH * N_FIELDS))
    return inter_half[:, None] + linear_term


# speedup vs baseline: 1.0022x; 1.0022x over previous
import jax
import jax.numpy as jnp
from jax import lax
from jax.experimental import pallas as pl
from jax.experimental.pallas import tpu as pltpu
from jax.experimental.pallas import tpu_sc as plsc

BATCH = 16384
N_FIELDS = 26
NUM_FACTORS = 16

NC = 2
NS = 16
NW = NC * NS
ROWS_PER_W = BATCH // NW
IDX_PER_W = ROWS_PER_W * N_FIELDS
GATHER_W = 128
IDX_ROWS_PER_W = IDX_PER_W // GATHER_W
CHUNK_ROWS = 64
CHUNK_IDX = CHUNK_ROWS * N_FIELDS
DMAS_PER_CHUNK = CHUNK_IDX // GATHER_W
N_CHUNKS = ROWS_PER_W // CHUNK_ROWS


def _fm_body(x_hbm, emb_hbm, fc_hbm, out_hbm, fcout_hbm,
             idx_v, emb_buf, fc_buf, out_v, fcvec_v, sem):
    wid = lax.axis_index("s") * NC + lax.axis_index("c")

    pltpu.sync_copy(x_hbm.at[pl.ds(wid * IDX_ROWS_PER_W, IDX_ROWS_PER_W), :],
                    idx_v)

    def start_chunk(g, ebuf, fbuf):
        handles = []
        for j in range(DMAS_PER_CHUNK):
            isl = idx_v.at[g * DMAS_PER_CHUNK + j]
            handles.append(pltpu.async_copy(
                emb_hbm.at[isl], ebuf.at[pl.ds(j * GATHER_W, GATHER_W), :], sem))
            handles.append(pltpu.async_copy(
                fc_hbm.at[isl], fbuf.at[pl.ds(j * GATHER_W, GATHER_W)], sem))
        return handles

    lane_ids = lax.iota(jnp.int32, NUM_FACTORS)

    def hsum(v):
        for k in (8, 4, 2, 1):
            v = v + v.at[lane_ids ^ k].get(mode="promise_in_bounds")
        return v

    def compute_chunk(g, ebuf, fbuf, fc_acc):
        out_base = g * CHUNK_ROWS

        def pair_body(i, acc):
            r = 2 * i
            inters = []
            for dr in (0, 1):
                rbase = (r + dr) * N_FIELDS
                s = ebuf[rbase, :]
                sq = s * s
                for j in range(1, N_FIELDS):
                    v = ebuf[rbase + j, :]
                    s = s + v
                    sq = sq + v * v
                inters.append(s * s - sq)
            h0, h1 = hsum(inters[0]), hsum(inters[1])
            lane = r % NUM_FACTORS
            acc = jnp.where(lane_ids == lane, 0.5 * h0, acc)
            acc = jnp.where(lane_ids == lane + 1, 0.5 * h1, acc)

            @pl.when(lane == NUM_FACTORS - 2)
            def _flush():
                out_v[pl.ds(out_base + r - (NUM_FACTORS - 2), NUM_FACTORS)] = acc

            return acc

        lax.fori_loop(0, CHUNK_ROWS // 2, pair_body,
                      jnp.zeros((NUM_FACTORS,), jnp.float32))

        return lax.fori_loop(
            0, CHUNK_IDX // NUM_FACTORS,
            lambda k, a: a + fbuf[pl.ds(k * NUM_FACTORS, NUM_FACTORS)],
            fc_acc)

    fc_acc = jnp.zeros((NUM_FACTORS,), jnp.float32)
    handles = start_chunk(0, emb_buf.at[0], fc_buf.at[0])
    for g in range(N_CHUNKS):
        p = g % 2
        for h in handles:
            h.wait()
        if g + 1 < N_CHUNKS:
            handles = start_chunk(g + 1, emb_buf.at[1 - p], fc_buf.at[1 - p])
        else:
            handles = []
        fc_acc = compute_chunk(g, emb_buf.at[p], fc_buf.at[p], fc_acc)

    fcvec_v[:] = fc_acc
    pltpu.sync_copy(out_v, out_hbm.at[pl.ds(wid * ROWS_PER_W, ROWS_PER_W)])
    pltpu.sync_copy(fcvec_v, fcout_hbm.at[wid])


def kernel(X, emb_table, fc_table, dense_W, dense_b):
    x_flat = X.astype(jnp.int32).reshape(BATCH * N_FIELDS // GATHER_W, GATHER_W)
    fc_flat = fc_table.reshape(-1)

    mesh = plsc.VectorSubcoreMesh(core_axis_name="c", subcore_axis_name="s")
    fm = pl.kernel(
        _fm_body,
        mesh=mesh,
        compiler_params=pltpu.CompilerParams(use_tc_tiling_on_sc=False),
        out_type=[
            jax.ShapeDtypeStruct((BATCH,), jnp.float32),
            jax.ShapeDtypeStruct((NW, NUM_FACTORS), jnp.float32),
        ],
        scratch_types=[
            pltpu.VMEM((IDX_ROWS_PER_W, GATHER_W), jnp.int32),
            pltpu.VMEM((2, CHUNK_IDX, NUM_FACTORS), jnp.float32),
            pltpu.VMEM((2, CHUNK_IDX), jnp.float32),
            pltpu.VMEM((ROWS_PER_W,), jnp.float32),
            pltpu.VMEM((NUM_FACTORS,), jnp.float32),
            pltpu.SemaphoreType.DMA,
        ],
    )
    inter_half, fc_parts = fm(x_flat, emb_table, fc_flat)

    linear_term = (dense_W[0, 0] * jnp.sum(fc_parts)
                   + dense_b[0] * (BATCH * N_FIELDS))
    return inter_half[:, None] + linear_term


# speedup vs baseline: 1.0346x; 1.0324x over previous
import jax
import jax.numpy as jnp
from jax import lax
from jax.experimental import pallas as pl
from jax.experimental.pallas import tpu as pltpu
from jax.experimental.pallas import tpu_sc as plsc

BATCH = 16384
N_FIELDS = 26
NUM_FACTORS = 16

NC = 2
NS = 16
NW = NC * NS
ROWS_PER_W = BATCH // NW
IDX_PER_W = ROWS_PER_W * N_FIELDS
GATHER_W = 128
IDX_ROWS_PER_W = IDX_PER_W // GATHER_W
CHUNK_ROWS = 64
CHUNK_IDX = CHUNK_ROWS * N_FIELDS
DMAS_PER_CHUNK = CHUNK_IDX // GATHER_W
N_CHUNKS = ROWS_PER_W // CHUNK_ROWS


def _fm_body(x_hbm, emb_hbm, out_hbm, idx_v, emb_buf, out_v, sem):
    wid = lax.axis_index("s") * NC + lax.axis_index("c")

    pltpu.sync_copy(x_hbm.at[pl.ds(wid * IDX_ROWS_PER_W, IDX_ROWS_PER_W), :],
                    idx_v)

    def start_chunk(g, ebuf):
        handles = []
        for j in range(DMAS_PER_CHUNK):
            isl = idx_v.at[g * DMAS_PER_CHUNK + j]
            handles.append(pltpu.async_copy(
                emb_hbm.at[isl], ebuf.at[pl.ds(j * GATHER_W, GATHER_W), :], sem))
        return handles

    lane_ids = lax.iota(jnp.int32, NUM_FACTORS)

    def hsum(v):
        for k in (8, 4, 2, 1):
            v = v + v.at[lane_ids ^ k].get(mode="promise_in_bounds")
        return v

    def compute_chunk(g, ebuf):
        out_base = g * CHUNK_ROWS

        def pair_body(i, acc):
            r = 2 * i
            inters = []
            for dr in (0, 1):
                rbase = (r + dr) * N_FIELDS
                s = ebuf[rbase, :]
                sq = s * s
                for j in range(1, N_FIELDS):
                    v = ebuf[rbase + j, :]
                    s = s + v
                    sq = sq + v * v
                inters.append(s * s - sq)
            h0, h1 = hsum(inters[0]), hsum(inters[1])
            lane = r % NUM_FACTORS
            acc = jnp.where(lane_ids == lane, 0.5 * h0, acc)
            acc = jnp.where(lane_ids == lane + 1, 0.5 * h1, acc)

            @pl.when(lane == NUM_FACTORS - 2)
            def _flush():
                out_v[pl.ds(out_base + r - (NUM_FACTORS - 2), NUM_FACTORS)] = acc

            return acc

        lax.fori_loop(0, CHUNK_ROWS // 2, pair_body,
                      jnp.zeros((NUM_FACTORS,), jnp.float32))

    handles = start_chunk(0, emb_buf.at[0])
    for g in range(N_CHUNKS):
        p = g % 2
        for h in handles:
            h.wait()
        if g + 1 < N_CHUNKS:
            handles = start_chunk(g + 1, emb_buf.at[1 - p])
        else:
            handles = []
        compute_chunk(g, emb_buf.at[p])

    pltpu.sync_copy(out_v, out_hbm.at[pl.ds(wid * ROWS_PER_W, ROWS_PER_W)])


def _fc_body(x_hbm, fc_hbm, fcout_hbm, idx_v, fc_buf, fcvec_v, sem):
    wid = lax.axis_index("s") * NC + lax.axis_index("c")

    pltpu.sync_copy(x_hbm.at[pl.ds(wid * IDX_ROWS_PER_W, IDX_ROWS_PER_W), :],
                    idx_v)

    def start_chunk(g, fbuf):
        handles = []
        for j in range(DMAS_PER_CHUNK):
            isl = idx_v.at[g * DMAS_PER_CHUNK + j]
            handles.append(pltpu.async_copy(
                fc_hbm.at[isl], fbuf.at[pl.ds(j * GATHER_W, GATHER_W)], sem))
        return handles

    def accum_chunk(fbuf, fc_acc):
        return lax.fori_loop(
            0, CHUNK_IDX // NUM_FACTORS,
            lambda k, a: a + fbuf[pl.ds(k * NUM_FACTORS, NUM_FACTORS)],
            fc_acc)

    fc_acc = jnp.zeros((NUM_FACTORS,), jnp.float32)
    handles = start_chunk(0, fc_buf.at[0])
    for g in range(N_CHUNKS):
        p = g % 2
        for h in handles:
            h.wait()
        if g + 1 < N_CHUNKS:
            handles = start_chunk(g + 1, fc_buf.at[1 - p])
        else:
            handles = []
        fc_acc = accum_chunk(fc_buf.at[p], fc_acc)

    fcvec_v[:] = fc_acc
    pltpu.sync_copy(fcvec_v, fcout_hbm.at[wid])


def kernel(X, emb_table, fc_table, dense_W, dense_b):
    x_flat = X.astype(jnp.int32).reshape(BATCH * N_FIELDS // GATHER_W, GATHER_W)
    fc_flat = fc_table.reshape(-1)

    mesh = plsc.VectorSubcoreMesh(core_axis_name="c", subcore_axis_name="s")
    fm = pl.kernel(
        _fm_body,
        mesh=mesh,
        compiler_params=pltpu.CompilerParams(use_tc_tiling_on_sc=False),
        out_type=[
            jax.ShapeDtypeStruct((BATCH,), jnp.float32),
        ],
        scratch_types=[
            pltpu.VMEM((IDX_ROWS_PER_W, GATHER_W), jnp.int32),
            pltpu.VMEM((2, CHUNK_IDX, NUM_FACTORS), jnp.float32),
            pltpu.VMEM((ROWS_PER_W,), jnp.float32),
            pltpu.SemaphoreType.DMA,
        ],
    )
    fc = pl.kernel(
        _fc_body,
        mesh=mesh,
        compiler_params=pltpu.CompilerParams(use_tc_tiling_on_sc=False),
        out_type=[
            jax.ShapeDtypeStruct((NW, NUM_FACTORS), jnp.float32),
        ],
        scratch_types=[
            pltpu.VMEM((IDX_ROWS_PER_W, GATHER_W), jnp.int32),
            pltpu.VMEM((2, CHUNK_IDX), jnp.float32),
            pltpu.VMEM((NUM_FACTORS,), jnp.float32),
            pltpu.SemaphoreType.DMA,
        ],
    )
    (inter_half,) = fm(x_flat, emb_table)
    (fc_parts,) = fc(x_flat, fc_flat)

    linear_term = (dense_W[0, 0] * jnp.sum(fc_parts)
                   + dense_b[0] * (BATCH * N_FIELDS))
    return inter_half[:, None] + linear_term
